# baseline (speedup 1.0000x reference)
import functools

import jax
import jax.numpy as jnp
from jax import lax
from jax.experimental import pallas as pl
from jax.experimental.pallas import tpu as pltpu
from jax.experimental.pallas import tpu_sc as plsc

_N = 10000
_E = 160000
_DF = 128
_DE = 4
_OC = 8
_IT = 4

_NT = 16
_EPT = _E // _NT
_C = 80
_NCH = _EPT // _C
_NPT = _N // _NT
_WPT = _NPT * _OC
_SENT = -1.0e30

_GDN = lax.GatherDimensionNumbers(
    offset_dims=(), collapsed_slice_dims=(0,), start_index_map=(0,))


def _vg(v, idx):
    return lax.gather(v, idx[:, None], _GDN, (1,),
                      mode=lax.GatherScatterMode.PROMISE_IN_BOUNDS)


def _prep_body(x_ref, a_ref, b_ref, o_ref):
    o_ref[...] = (
        jnp.dot(x_ref[...], a_ref[...], preferred_element_type=jnp.float32,
                precision=lax.Precision.HIGHEST)
        + b_ref[...])


def _tc_prep(x, a56, bias56):
    return pl.pallas_call(
        _prep_body,
        grid=(10,),
        in_specs=[
            pl.BlockSpec((_N // 10, _DF), lambda i: (i, 0)),
            pl.BlockSpec((_DF, 56), lambda i: (0, 0)),
            pl.BlockSpec((1, 56), lambda i: (0, 0)),
        ],
        out_specs=pl.BlockSpec((_N // 10, 56), lambda i: (i, 0)),
        out_shape=jax.ShapeDtypeStruct((_N, 56), jnp.float32),
    )(x, a56, bias56)


def _readout_body(h_ref, w1_ref, r8_ref, s_ref, b1_ref, w2_ref, b2_ref,
                  o_ref, acc_ref):
    i = pl.program_id(0)

    @pl.when(i == 0)
    def _():
        acc_ref[...] = jnp.zeros_like(acc_ref)

    hrep = jnp.dot(h_ref[...], r8_ref[...],
                   preferred_element_type=jnp.float32,
                   precision=lax.Precision.HIGHEST)
    acc_ref[...] += jnp.sum(hrep * w1_ref[...], axis=0, keepdims=True)

    @pl.when(i == pl.num_programs(0) - 1)
    def _():
        s = jnp.dot(acc_ref[...], s_ref[...],
                    preferred_element_type=jnp.float32,
                    precision=lax.Precision.HIGHEST) + b1_ref[...]
        s = jnp.where(s > 0, s, jnp.exp(s) - 1.0)
        o = jnp.dot(s, w2_ref[...],
                    preferred_element_type=jnp.float32,
                    precision=lax.Precision.HIGHEST) + b2_ref[...]
        o_ref[...] = jnp.where(o > 0, o, jnp.exp(o) - 1.0)


def _tc_readout(h2d, w1r, r8, smat, b1, w2, b2):
    return pl.pallas_call(
        _readout_body,
        grid=(10,),
        in_specs=[
            pl.BlockSpec((_N // 10, _OC), lambda i: (i, 0)),
            pl.BlockSpec((_N // 10, 128), lambda i: (i, 0)),
            pl.BlockSpec((_OC, 128), lambda i: (0, 0)),
            pl.BlockSpec((128, 16), lambda i: (0, 0)),
            pl.BlockSpec((1, 16), lambda i: (0, 0)),
            pl.BlockSpec((16, 1), lambda i: (0, 0)),
            pl.BlockSpec((1, 1), lambda i: (0, 0)),
        ],
        out_specs=pl.BlockSpec((1, 1), lambda i: (0, 0)),
        out_shape=jax.ShapeDtypeStruct((1, 1), jnp.float32),
        scratch_shapes=[pltpu.VMEM((1, 128), jnp.float32)],
    )(h2d, w1r, r8, smat, b1, w2, b2)


def _gnn_body(src_h, dst_h, ea_h, z0_h, b0_h, wz_h, rd_h, bd_h,
              hout_h,
              zs_h, stage, acc, srcv, dstv, eav, zrv,
              mrg, tmp, hb, bb, zbuf, wzv, rdv, bdv, sem):
    wid = lax.axis_index("s")
    ebase = wid * _EPT
    nodew = wid * _WPT
    nbase = wid * _NPT

    pltpu.sync_copy(wz_h, wzv)
    pltpu.sync_copy(rd_h, rdv)
    pltpu.sync_copy(bd_h, bdv)

    lane = lax.iota(jnp.int32, 16)
    lo8 = lane < 8
    idx01 = lane >> 3
    idx23 = idx01 + 2
    swap = lane ^ 8
    sentv = jnp.full((16,), _SENT, jnp.float32)

    def init_acc(i, c):
        acc[pl.ds(i * 16, 16)] = sentv
        return c

    lax.fori_loop(0, (_EPT * _OC + 16) // 16, init_acc, 0)

    for l in range(_IT + 1):
        zt = z0_h if l == 0 else zs_h

        def chunk_body(k, c):
            eb = ebase + k * _C
            pltpu.sync_copy(src_h.at[pl.ds(eb, _C)], srcv)
            pltpu.sync_copy(dst_h.at[pl.ds(eb, _C)], dstv)
            pltpu.sync_copy(ea_h.at[pl.ds(eb * 8, _C * 8)], eav)
            pltpu.async_copy(zt.at[srcv], zrv, sem).wait()

            def group_body(g, c2):
                d8vec = dstv[pl.ds(g * 16, 16)] * 8
                for j in range(16):
                    vea = eav[pl.ds(g * 128 + j * 8, 16)]
                    se01 = _vg(vea, idx01)
                    se23 = _vg(vea, idx23)
                    e = g * 16 + j
                    v01 = zrv[e, pl.ds(0, 16)]
                    v23 = zrv[e, pl.ds(16, 16)]
                    vb = zrv[e, pl.ds(32, 16)]
                    m = vb + se01 * v01 + se23 * v23
                    m = m + _vg(m, swap)
                    m = jnp.where(lo8, m, sentv)
                    d8 = d8vec[j]
                    acc[pl.ds(d8, 16)] = jnp.maximum(acc[pl.ds(d8, 16)], m)
                return c2

            return lax.fori_loop(0, _C // 16, group_body, c)

        lax.fori_loop(0, _NCH, chunk_body, 0)

        pltpu.sync_copy(acc.at[pl.ds(0, _N * _OC)],
                        stage.at[pl.ds(wid * (_N * _OC), _N * _OC)])
        plsc.subcore_barrier()

        if l == 0:
            pltpu.sync_copy(b0_h.at[pl.ds(nodew, _WPT)],
                            bb.at[pl.ds(0, _WPT)])

        nv = (_WPT + 15) // 16

        for t in range(_NT):
            pltpu.sync_copy(stage.at[pl.ds(t * (_N * _OC) + nodew, _WPT)],
                            tmp.at[pl.ds(0, _WPT)])
            if t == 0:
                def cp(i, c):
                    mrg[pl.ds(i * 16, 16)] = tmp[pl.ds(i * 16, 16)]
                    return c
                lax.fori_loop(0, nv, cp, 0)
            else:
                def mx(i, c):
                    mrg[pl.ds(i * 16, 16)] = jnp.maximum(
                        mrg[pl.ds(i * 16, 16)], tmp[pl.ds(i * 16, 16)])
                    return c
                lax.fori_loop(0, nv, mx, 0)

        def hfun(i, c):
            m = mrg[pl.ds(i * 16, 16)]
            a = jnp.where(m == _SENT, 0.0, m)
            hb[pl.ds(i * 16, 16)] = jnp.clip(
                a + bb[pl.ds(i * 16, 16)], 0.0, 6.0)
            return c

        lax.fori_loop(0, nv, hfun, 0)

        if l < _IT:
            def bfun(p, c):
                hv = hb[pl.ds(p * 16, 16)]
                b = bdv[l, pl.ds(0, 16)]
                for i in range(_OC):
                    b = b + _vg(hv, idx01 * 8 + i) * rdv[l, i, pl.ds(0, 16)]
                bb[pl.ds(p * 16, 16)] = b
                return c

            lax.fori_loop(0, nv, bfun, 0)

            for g in range(5):
                def zfun(r, c):
                    hv = hb[pl.ds((g * 125 + r) * 8, 16)]
                    a01 = jnp.zeros((16,), jnp.float32)
                    a23 = jnp.zeros((16,), jnp.float32)
                    ab = jnp.zeros((16,), jnp.float32)
                    for i in range(_OC):
                        s = _vg(hv, lane * 0 + i)
                        a01 = a01 + s * wzv[i, pl.ds(0, 16)]
                        a23 = a23 + s * wzv[i, pl.ds(16, 16)]
                        ab = ab + s * wzv[i, pl.ds(32, 16)]
                    zbuf[r, pl.ds(0, 16)] = a01
                    zbuf[r, pl.ds(16, 16)] = a23
                    zbuf[r, pl.ds(32, 16)] = ab
                    return c

                lax.fori_loop(0, 125, zfun, 0)
                pltpu.sync_copy(zbuf, zs_h.at[pl.ds(nbase + g * 125, 125)])

            lax.fori_loop(0, (_EPT * _OC + 16) // 16, init_acc, 0)
            plsc.subcore_barrier()
        else:
            pltpu.sync_copy(hb.at[pl.ds(0, _WPT)],
                            hout_h.at[pl.ds(nodew, _WPT)])


def _sc_gnn(src, dst, ea8, z0, b0, wz, rd, bd):
    mesh = plsc.VectorSubcoreMesh(
        core_axis_name="c", subcore_axis_name="s", num_cores=1)
    f = functools.partial(
        pl.kernel,
        out_type=jax.ShapeDtypeStruct((_N * _OC,), jnp.float32),
        mesh=mesh,
        compiler_params=pltpu.CompilerParams(use_tc_tiling_on_sc=False),
        scratch_types=[
            pltpu.HBM((_N, 48), jnp.float32),
            pltpu.HBM((_NT * _N * _OC,), jnp.float32),
            pltpu.VMEM((_EPT * _OC + 16,), jnp.float32),
            pltpu.VMEM((_C,), jnp.int32),
            pltpu.VMEM((_C,), jnp.int32),
            pltpu.VMEM((_C * 8,), jnp.float32),
            pltpu.VMEM((_C, 48), jnp.float32),
            pltpu.VMEM((_WPT + 16,), jnp.float32),
            pltpu.VMEM((_WPT + 16,), jnp.float32),
            pltpu.VMEM((_WPT + 16,), jnp.float32),
            pltpu.VMEM((_WPT + 16,), jnp.float32),
            pltpu.VMEM((125, 48), jnp.float32),
            pltpu.VMEM((_OC, 48), jnp.float32),
            pltpu.VMEM((_IT, _OC, 16), jnp.float32),
            pltpu.VMEM((_IT, 16), jnp.float32),
            pltpu.SemaphoreType.DMA,
        ],
    )(_gnn_body)
    return f(src, dst, ea8, z0, b0, wz, rd, bd)


def kernel(x, edge_index, edge_attr, W_fe, b_fe, W_e, b_e, root0, bias0,
           roots, biases, W_o1, b_o1, W_o2, b_o2):
    f32 = jnp.float32
    src = edge_index[0].astype(jnp.int32)
    dst = edge_index[1].astype(jnp.int32)
    ea8 = jnp.pad(edge_attr, ((0, 0), (0, 4))).reshape(-1)

    wfe_r = W_fe.reshape(_DE, _DF, _OC).transpose(1, 0, 2).reshape(_DF, 32)
    a56 = jnp.concatenate(
        [wfe_r, b_fe.reshape(_DF, _OC), jnp.zeros((_DF, 8), f32), root0],
        axis=1)
    bias56 = jnp.concatenate(
        [jnp.zeros((48,), f32), bias0]).reshape(1, 56)

    we_r = W_e.reshape(_DE, _OC, _OC).transpose(1, 0, 2).reshape(_OC, 32)
    wz = jnp.concatenate(
        [we_r, b_e.reshape(_OC, _OC), jnp.zeros((_OC, 8), f32)], axis=1)
    rd = jnp.concatenate([roots, roots], axis=2)
    bd = jnp.concatenate([biases, biases], axis=1)

    zb0 = _tc_prep(x, a56, bias56)
    z0 = zb0[:, :48]
    b0 = zb0[:, 48:].reshape(-1)

    h4 = _sc_gnn(src, dst, ea8, z0, b0, wz, rd, bd)

    w1r = W_o1.reshape(_N, 128)
    r8 = jnp.zeros((_OC, 128), f32)
    r8 = r8.at[jnp.arange(8).repeat(16), jnp.arange(128)].set(1.0)
    smat = jnp.zeros((128, 16), f32)
    smat = smat.at[jnp.arange(128), jnp.arange(128) % 16].set(1.0)
    out = _tc_readout(h4.reshape(_N, _OC), w1r, r8,
                      smat, b_o1.reshape(1, 16), W_o2, b_o2.reshape(1, 1))
    return out.reshape(1)


# speedup vs baseline: 1.3211x; 1.3211x over previous
import functools

import jax
import jax.numpy as jnp
from jax import lax
from jax.experimental import pallas as pl
from jax.experimental.pallas import tpu as pltpu
from jax.experimental.pallas import tpu_sc as plsc

_N = 10000
_E = 160000
_DF = 128
_DE = 4
_OC = 8
_IT = 4

_NT = 16
_EPT = _E // _NT
_C = 80
_NCH = _EPT // _C
_NPT = _N // _NT
_WPT = _NPT * _OC
_SENT = -1.0e30

_GDN = lax.GatherDimensionNumbers(
    offset_dims=(), collapsed_slice_dims=(0,), start_index_map=(0,))


def _vg(v, idx):
    return lax.gather(v, idx[:, None], _GDN, (1,),
                      mode=lax.GatherScatterMode.PROMISE_IN_BOUNDS)


def _prep_body(x_ref, a_ref, b_ref, o_ref):
    o_ref[...] = (
        jnp.dot(x_ref[...], a_ref[...], preferred_element_type=jnp.float32,
                precision=lax.Precision.HIGHEST)
        + b_ref[...])


def _tc_prep(x, a56, bias56):
    return pl.pallas_call(
        _prep_body,
        grid=(10,),
        in_specs=[
            pl.BlockSpec((_N // 10, _DF), lambda i: (i, 0)),
            pl.BlockSpec((_DF, 56), lambda i: (0, 0)),
            pl.BlockSpec((1, 56), lambda i: (0, 0)),
        ],
        out_specs=pl.BlockSpec((_N // 10, 56), lambda i: (i, 0)),
        out_shape=jax.ShapeDtypeStruct((_N, 56), jnp.float32),
    )(x, a56, bias56)


def _readout_body(h_ref, w1_ref, r8_ref, s_ref, b1_ref, w2_ref, b2_ref,
                  o_ref, acc_ref):
    i = pl.program_id(0)

    @pl.when(i == 0)
    def _():
        acc_ref[...] = jnp.zeros_like(acc_ref)

    hrep = jnp.dot(h_ref[...], r8_ref[...],
                   preferred_element_type=jnp.float32,
                   precision=lax.Precision.HIGHEST)
    acc_ref[...] += jnp.sum(hrep * w1_ref[...], axis=0, keepdims=True)

    @pl.when(i == pl.num_programs(0) - 1)
    def _():
        s = jnp.dot(acc_ref[...], s_ref[...],
                    preferred_element_type=jnp.float32,
                    precision=lax.Precision.HIGHEST) + b1_ref[...]
        s = jnp.where(s > 0, s, jnp.exp(s) - 1.0)
        o = jnp.dot(s, w2_ref[...],
                    preferred_element_type=jnp.float32,
                    precision=lax.Precision.HIGHEST) + b2_ref[...]
        o_ref[...] = jnp.where(o > 0, o, jnp.exp(o) - 1.0)


def _tc_readout(h2d, w1r, r8, smat, b1, w2, b2):
    return pl.pallas_call(
        _readout_body,
        grid=(10,),
        in_specs=[
            pl.BlockSpec((_N // 10, _OC), lambda i: (i, 0)),
            pl.BlockSpec((_N // 10, 128), lambda i: (i, 0)),
            pl.BlockSpec((_OC, 128), lambda i: (0, 0)),
            pl.BlockSpec((128, 16), lambda i: (0, 0)),
            pl.BlockSpec((1, 16), lambda i: (0, 0)),
            pl.BlockSpec((16, 1), lambda i: (0, 0)),
            pl.BlockSpec((1, 1), lambda i: (0, 0)),
        ],
        out_specs=pl.BlockSpec((1, 1), lambda i: (0, 0)),
        out_shape=jax.ShapeDtypeStruct((1, 1), jnp.float32),
        scratch_shapes=[pltpu.VMEM((1, 128), jnp.float32)],
    )(h2d, w1r, r8, smat, b1, w2, b2)


def _gnn_body(src_h, dst_h, ea_h, z0_h, b0_h, wz_h, rd_h, bd_h, sent_h,
              hout_h,
              zs_h, stage, acc,
              srcv0, srcv1, dstv0, dstv1, eav0, eav1, zrv0, zrv1,
              mrg, tmp, hb, bb, zbuf, wzv, rdv, bdv,
              semg0, semg1, semi):
    wid = lax.axis_index("s")
    ebase = wid * _EPT
    nodew = wid * _WPT
    nbase = wid * _NPT

    pltpu.sync_copy(wz_h, wzv)
    pltpu.sync_copy(rd_h, rdv)
    pltpu.sync_copy(bd_h, bdv)

    lane = lax.iota(jnp.int32, 16)
    lo8 = lane < 8
    idx01 = lane >> 3
    idx23 = idx01 + 2
    swap = lane ^ 8
    sentv = jnp.full((16,), _SENT, jnp.float32)

    pltpu.async_copy(sent_h, acc, semi)

    def load_idx(k, srcv, dstv, eav):
        eb = ebase + k * _C
        pltpu.sync_copy(src_h.at[pl.ds(eb, _C)], srcv)
        pltpu.sync_copy(dst_h.at[pl.ds(eb, _C)], dstv)
        pltpu.sync_copy(ea_h.at[pl.ds(eb * 4, _C * 4)],
                        eav.at[pl.ds(0, _C * 4)])

    for g in range(5):
        pltpu.sync_copy(z0_h.at[pl.ds(nbase + g * 125, 125)], zbuf)
        pltpu.sync_copy(zbuf, zs_h.at[pl.ds(nbase + g * 125, 125)])
    plsc.subcore_barrier()

    def layer_body(l, carry):
        zt = zs_h

        def compute_chunk(dstv, eav, zrv):
            def group_body(g, c2):
                d8vec = dstv[pl.ds(g * 16, 16)] * 8
                for j in range(16):
                    vea = eav[pl.ds(g * 64 + j * 4, 16)]
                    se01 = _vg(vea, idx01)
                    se23 = _vg(vea, idx23)
                    e = g * 16 + j
                    v01 = zrv[e, pl.ds(0, 16)]
                    v23 = zrv[e, pl.ds(16, 16)]
                    vb = zrv[e, pl.ds(32, 16)]
                    m = vb + se01 * v01 + se23 * v23
                    m = m + _vg(m, swap)
                    m = jnp.where(lo8, m, sentv)
                    d8 = d8vec[j]
                    acc[pl.ds(d8, 16)] = jnp.maximum(acc[pl.ds(d8, 16)], m)
                return c2

            lax.fori_loop(0, _C // 16, group_body, 0)

        load_idx(0, srcv0, dstv0, eav0)
        pltpu.async_copy(zt.at[srcv0], zrv0, semg0)
        load_idx(1, srcv1, dstv1, eav1)
        pltpu.async_copy(zt.at[srcv1], zrv1, semg1)
        pltpu.make_async_copy(sent_h, acc, semi).wait()

        def pair_body(p, c):
            k0 = 2 * p
            pltpu.make_async_copy(zt.at[srcv0], zrv0, semg0).wait()
            compute_chunk(dstv0, eav0, zrv0)

            @pl.when(k0 + 2 < _NCH)
            def _():
                load_idx(k0 + 2, srcv0, dstv0, eav0)
                pltpu.async_copy(zt.at[srcv0], zrv0, semg0)

            @pl.when(k0 + 1 < _NCH)
            def _():
                pltpu.make_async_copy(zt.at[srcv1], zrv1, semg1).wait()
                compute_chunk(dstv1, eav1, zrv1)

            @pl.when(k0 + 3 < _NCH)
            def _():
                load_idx(k0 + 3, srcv1, dstv1, eav1)
                pltpu.async_copy(zt.at[srcv1], zrv1, semg1)

            return c

        lax.fori_loop(0, (_NCH + 1) // 2, pair_body, 0)

        pltpu.sync_copy(acc.at[pl.ds(0, _N * _OC)],
                        stage.at[pl.ds(wid * (_N * _OC), _N * _OC)])
        plsc.subcore_barrier()

        @pl.when(l < _IT)
        def _():
            pltpu.async_copy(sent_h, acc, semi)

        @pl.when(l == 0)
        def _():
            pltpu.sync_copy(b0_h.at[pl.ds(nodew, _WPT)],
                            bb.at[pl.ds(0, _WPT)])

        nv = (_WPT + 15) // 16

        pltpu.sync_copy(stage.at[pl.ds(nodew, _WPT)], tmp.at[pl.ds(0, _WPT)])

        def cp(i, c):
            mrg[pl.ds(i * 16, 16)] = tmp[pl.ds(i * 16, 16)]
            return c

        lax.fori_loop(0, nv, cp, 0)

        def tmerge(t, c):
            pltpu.sync_copy(stage.at[pl.ds(t * (_N * _OC) + nodew, _WPT)],
                            tmp.at[pl.ds(0, _WPT)])

            def mx(i, c2):
                mrg[pl.ds(i * 16, 16)] = jnp.maximum(
                    mrg[pl.ds(i * 16, 16)], tmp[pl.ds(i * 16, 16)])
                return c2

            return lax.fori_loop(0, nv, mx, c)

        lax.fori_loop(1, _NT, tmerge, 0)

        def hfun(i, c):
            m = mrg[pl.ds(i * 16, 16)]
            a = jnp.where(m == _SENT, 0.0, m)
            hb[pl.ds(i * 16, 16)] = jnp.clip(
                a + bb[pl.ds(i * 16, 16)], 0.0, 6.0)
            return c

        lax.fori_loop(0, nv, hfun, 0)

        lm = jnp.minimum(l, _IT - 1)

        @pl.when(l < _IT)
        def _():
            def bfun(p, c):
                hv = hb[pl.ds(p * 16, 16)]
                b = bdv[lm, pl.ds(0, 16)]
                for i in range(_OC):
                    b = b + _vg(hv, idx01 * 8 + i) * rdv[lm, i, pl.ds(0, 16)]
                bb[pl.ds(p * 16, 16)] = b
                return c

            lax.fori_loop(0, nv, bfun, 0)

            for g in range(5):
                def zfun(r, c):
                    hv = hb[pl.ds((g * 125 + r) * 8, 16)]
                    a01 = jnp.zeros((16,), jnp.float32)
                    a23 = jnp.zeros((16,), jnp.float32)
                    ab = jnp.zeros((16,), jnp.float32)
                    for i in range(_OC):
                        s = _vg(hv, lane * 0 + i)
                        a01 = a01 + s * wzv[i, pl.ds(0, 16)]
                        a23 = a23 + s * wzv[i, pl.ds(16, 16)]
                        ab = ab + s * wzv[i, pl.ds(32, 16)]
                    zbuf[r, pl.ds(0, 16)] = a01
                    zbuf[r, pl.ds(16, 16)] = a23
                    zbuf[r, pl.ds(32, 16)] = ab
                    return c

                lax.fori_loop(0, 125, zfun, 0)
                pltpu.sync_copy(zbuf, zs_h.at[pl.ds(nbase + g * 125, 125)])

            plsc.subcore_barrier()

        @pl.when(l == _IT)
        def _():
            pltpu.sync_copy(hb.at[pl.ds(0, _WPT)],
                            hout_h.at[pl.ds(nodew, _WPT)])

        return carry

    lax.fori_loop(0, _IT + 1, layer_body, 0)


def _sc_gnn(src, dst, ea4, z0, b0, wz, rd, bd, sent):
    mesh = plsc.VectorSubcoreMesh(
        core_axis_name="c", subcore_axis_name="s", num_cores=1)
    f = functools.partial(
        pl.kernel,
        out_type=jax.ShapeDtypeStruct((_N * _OC,), jnp.float32),
        mesh=mesh,
        compiler_params=pltpu.CompilerParams(use_tc_tiling_on_sc=False),
        scratch_types=[
            pltpu.HBM((_N, 48), jnp.float32),
            pltpu.HBM((_NT * _N * _OC,), jnp.float32),
            pltpu.VMEM((_EPT * _OC + 16,), jnp.float32),
            pltpu.VMEM((_C,), jnp.int32),
            pltpu.VMEM((_C,), jnp.int32),
            pltpu.VMEM((_C,), jnp.int32),
            pltpu.VMEM((_C,), jnp.int32),
            pltpu.VMEM((_C * 4 + 16,), jnp.float32),
            pltpu.VMEM((_C * 4 + 16,), jnp.float32),
            pltpu.VMEM((_C, 48), jnp.float32),
            pltpu.VMEM((_C, 48), jnp.float32),
            pltpu.VMEM((_WPT + 16,), jnp.float32),
            pltpu.VMEM((_WPT + 16,), jnp.float32),
            pltpu.VMEM((_WPT + 16,), jnp.float32),
            pltpu.VMEM((_WPT + 16,), jnp.float32),
            pltpu.VMEM((125, 48), jnp.float32),
            pltpu.VMEM((_OC, 48), jnp.float32),
            pltpu.VMEM((_IT, _OC, 16), jnp.float32),
            pltpu.VMEM((_IT, 16), jnp.float32),
            pltpu.SemaphoreType.DMA,
            pltpu.SemaphoreType.DMA,
            pltpu.SemaphoreType.DMA,
        ],
    )(_gnn_body)
    return f(src, dst, ea4, z0, b0, wz, rd, bd, sent)


def kernel(x, edge_index, edge_attr, W_fe, b_fe, W_e, b_e, root0, bias0,
           roots, biases, W_o1, b_o1, W_o2, b_o2):
    f32 = jnp.float32
    src = edge_index[0].astype(jnp.int32)
    dst = edge_index[1].astype(jnp.int32)
    ea4 = edge_attr.reshape(-1)
    sent = jnp.full((_N * _OC + 16,), _SENT, f32)

    wfe_r = W_fe.reshape(_DE, _DF, _OC).transpose(1, 0, 2).reshape(_DF, 32)
    a56 = jnp.concatenate(
        [wfe_r, b_fe.reshape(_DF, _OC), jnp.zeros((_DF, 8), f32), root0],
        axis=1)
    bias56 = jnp.concatenate(
        [jnp.zeros((48,), f32), bias0]).reshape(1, 56)

    we_r = W_e.reshape(_DE, _OC, _OC).transpose(1, 0, 2).reshape(_OC, 32)
    wz = jnp.concatenate(
        [we_r, b_e.reshape(_OC, _OC), jnp.zeros((_OC, 8), f32)], axis=1)
    rd = jnp.concatenate([roots, roots], axis=2)
    bd = jnp.concatenate([biases, biases], axis=1)

    zb0 = _tc_prep(x, a56, bias56)
    z0 = zb0[:, :48]
    b0 = zb0[:, 48:].reshape(-1)

    h4 = _sc_gnn(src, dst, ea4, z0, b0, wz, rd, bd, sent)

    w1r = W_o1.reshape(_N, 128)
    r8 = jnp.zeros((_OC, 128), f32)
    r8 = r8.at[jnp.arange(8).repeat(16), jnp.arange(128)].set(1.0)
    smat = jnp.zeros((128, 16), f32)
    smat = smat.at[jnp.arange(128), jnp.arange(128) % 16].set(1.0)
    out = _tc_readout(h4.reshape(_N, _OC), w1r, r8,
                      smat, b_o1.reshape(1, 16), W_o2, b_o2.reshape(1, 1))
    return out.reshape(1)


# speedup vs baseline: 1.7946x; 1.3584x over previous
import functools

import jax
import jax.numpy as jnp
from jax import lax
from jax.experimental import pallas as pl
from jax.experimental.pallas import tpu as pltpu
from jax.experimental.pallas import tpu_sc as plsc

_N = 10000
_E = 160000
_DF = 128
_DE = 4
_OC = 8
_IT = 4

_NT = 16
_EPT = _E // _NT
_C = 80
_NCH = _EPT // _C
_NPT = _N // _NT
_WPT = _NPT * _OC
_SENT = -1.0e30

_GDN = lax.GatherDimensionNumbers(
    offset_dims=(), collapsed_slice_dims=(0,), start_index_map=(0,))


def _vg(v, idx):
    return lax.gather(v, idx[:, None], _GDN, (1,),
                      mode=lax.GatherScatterMode.PROMISE_IN_BOUNDS)


def _prep_body(x_ref, a_ref, b_ref, o_ref):
    o_ref[...] = (
        jnp.dot(x_ref[...], a_ref[...], preferred_element_type=jnp.float32,
                precision=lax.Precision.HIGHEST)
        + b_ref[...])


def _tc_prep(x, a56, bias56):
    return pl.pallas_call(
        _prep_body,
        grid=(10,),
        in_specs=[
            pl.BlockSpec((_N // 10, _DF), lambda i: (i, 0)),
            pl.BlockSpec((_DF, 56), lambda i: (0, 0)),
            pl.BlockSpec((1, 56), lambda i: (0, 0)),
        ],
        out_specs=pl.BlockSpec((_N // 10, 56), lambda i: (i, 0)),
        out_shape=jax.ShapeDtypeStruct((_N, 56), jnp.float32),
    )(x, a56, bias56)


def _readout_body(h_ref, w1_ref, r8_ref, s_ref, b1_ref, w2_ref, b2_ref,
                  o_ref, acc_ref):
    i = pl.program_id(0)

    @pl.when(i == 0)
    def _():
        acc_ref[...] = jnp.zeros_like(acc_ref)

    hrep = jnp.dot(h_ref[...], r8_ref[...],
                   preferred_element_type=jnp.float32,
                   precision=lax.Precision.HIGHEST)
    acc_ref[...] += jnp.sum(hrep * w1_ref[...], axis=0, keepdims=True)

    @pl.when(i == pl.num_programs(0) - 1)
    def _():
        s = jnp.dot(acc_ref[...], s_ref[...],
                    preferred_element_type=jnp.float32,
                    precision=lax.Precision.HIGHEST) + b1_ref[...]
        s = jnp.where(s > 0, s, jnp.exp(s) - 1.0)
        o = jnp.dot(s, w2_ref[...],
                    preferred_element_type=jnp.float32,
                    precision=lax.Precision.HIGHEST) + b2_ref[...]
        o_ref[...] = jnp.where(o > 0, o, jnp.exp(o) - 1.0)


def _tc_readout(h2d, w1r, r8, smat, b1, w2, b2):
    return pl.pallas_call(
        _readout_body,
        grid=(10,),
        in_specs=[
            pl.BlockSpec((_N // 10, _OC), lambda i: (i, 0)),
            pl.BlockSpec((_N // 10, 128), lambda i: (i, 0)),
            pl.BlockSpec((_OC, 128), lambda i: (0, 0)),
            pl.BlockSpec((128, 16), lambda i: (0, 0)),
            pl.BlockSpec((1, 16), lambda i: (0, 0)),
            pl.BlockSpec((16, 1), lambda i: (0, 0)),
            pl.BlockSpec((1, 1), lambda i: (0, 0)),
        ],
        out_specs=pl.BlockSpec((1, 1), lambda i: (0, 0)),
        out_shape=jax.ShapeDtypeStruct((1, 1), jnp.float32),
        scratch_shapes=[pltpu.VMEM((1, 128), jnp.float32)],
    )(h2d, w1r, r8, smat, b1, w2, b2)


def _gnn_body(src_h, dst_h, ea_h, z0_h, b0_h, wz_h, rd_h, bd_h, sent_h,
              hout_h,
              zs_h, stage, acc,
              srcv0, srcv1, srcv2, srcv3,
              dstv0, dstv1, dstv2, dstv3,
              eav0, eav1, eav2, eav3, zrv0, zrv1,
              mrg, tmp, hb, bb, zbuf, wzv, rdv, bdv,
              semg0, semg1, semi, si0, si1, si2, si3):
    wid = lax.axis_index("s")
    ebase = wid * _EPT
    nodew = wid * _WPT
    nbase = wid * _NPT

    pltpu.sync_copy(wz_h, wzv)
    pltpu.sync_copy(rd_h, rdv)
    pltpu.sync_copy(bd_h, bdv)

    lane = lax.iota(jnp.int32, 16)
    lo8 = lane < 8
    idx01 = lane >> 3
    idx23 = idx01 + 2
    swap = lane ^ 8
    sentv = jnp.full((16,), _SENT, jnp.float32)

    pltpu.async_copy(sent_h, acc, semi)

    def load_idx(k, srcv, dstv, eav, sem):
        eb = ebase + k * _C
        pltpu.async_copy(src_h.at[pl.ds(eb, _C)], srcv, sem)
        pltpu.async_copy(dst_h.at[pl.ds(eb, _C)], dstv, sem)
        pltpu.async_copy(ea_h.at[pl.ds(eb * 4, _C * 4)],
                         eav.at[pl.ds(0, _C * 4)], sem)

    def wait_idx(srcv, dstv, eav, sem):
        pltpu.make_async_copy(src_h.at[pl.ds(0, _C)], srcv, sem).wait()
        pltpu.make_async_copy(dst_h.at[pl.ds(0, _C)], dstv, sem).wait()
        pltpu.make_async_copy(ea_h.at[pl.ds(0, _C * 4)],
                              eav.at[pl.ds(0, _C * 4)], sem).wait()

    for g in range(5):
        pltpu.sync_copy(z0_h.at[pl.ds(nbase + g * 125, 125)], zbuf)
        pltpu.sync_copy(zbuf, zs_h.at[pl.ds(nbase + g * 125, 125)])
    plsc.subcore_barrier()

    def layer_body(l, carry):
        zt = zs_h

        def compute_chunk(dstv, eav, zrv):
            def group_body(g, c2):
                d8vec = dstv[pl.ds(g * 16, 16)] * 8
                for j in range(16):
                    vea = eav[pl.ds(g * 64 + j * 4, 16)]
                    se01 = _vg(vea, idx01)
                    se23 = _vg(vea, idx23)
                    e = g * 16 + j
                    v01 = zrv[e, pl.ds(0, 16)]
                    v23 = zrv[e, pl.ds(16, 16)]
                    vb = zrv[e, pl.ds(32, 16)]
                    m = vb + se01 * v01 + se23 * v23
                    m = m + _vg(m, swap)
                    m = jnp.where(lo8, m, sentv)
                    d8 = d8vec[j]
                    acc[pl.ds(d8, 16)] = jnp.maximum(acc[pl.ds(d8, 16)], m)
                return c2

            lax.fori_loop(0, _C // 16, group_body, 0)

        load_idx(0, srcv0, dstv0, eav0, si0)
        wait_idx(srcv0, dstv0, eav0, si0)
        load_idx(1, srcv1, dstv1, eav1, si1)
        wait_idx(srcv1, dstv1, eav1, si1)
        pltpu.async_copy(zt.at[srcv0], zrv0, semg0)
        pltpu.async_copy(zt.at[srcv1], zrv1, semg1)
        load_idx(2, srcv2, dstv2, eav2, si2)
        load_idx(3, srcv3, dstv3, eav3, si3)
        pltpu.make_async_copy(sent_h, acc, semi).wait()

        idx_sets = ((srcv0, dstv0, eav0, si0), (srcv1, dstv1, eav1, si1),
                    (srcv2, dstv2, eav2, si2), (srcv3, dstv3, eav3, si3))

        def quad_body(q, c):
            k0 = 4 * q
            for s in range(4):
                k = k0 + s
                srcv, dstv, eav, si = idx_sets[s]
                nsrcv, ndstv, neav, nsi = idx_sets[(s + 2) % 4]
                zrv, semg = (zrv0, semg0) if s % 2 == 0 else (zrv1, semg1)

                if s == 0:
                    pltpu.make_async_copy(zt.at[srcv], zrv, semg).wait()
                    compute_chunk(dstv, eav, zrv)
                else:
                    @pl.when(k < _NCH)
                    def _():
                        pltpu.make_async_copy(zt.at[srcv], zrv, semg).wait()
                        compute_chunk(dstv, eav, zrv)

                @pl.when(k + 2 < _NCH)
                def _():
                    wait_idx(nsrcv, ndstv, neav, nsi)
                    pltpu.async_copy(zt.at[nsrcv], zrv, semg)

                @pl.when(k + 4 < _NCH)
                def _():
                    load_idx(k + 4, srcv, dstv, eav, si)

            return c

        lax.fori_loop(0, (_NCH + 3) // 4, quad_body, 0)

        pltpu.sync_copy(acc.at[pl.ds(0, _N * _OC)],
                        stage.at[pl.ds(wid * (_N * _OC), _N * _OC)])
        plsc.subcore_barrier()

        @pl.when(l < _IT)
        def _():
            pltpu.async_copy(sent_h, acc, semi)

        @pl.when(l == 0)
        def _():
            pltpu.sync_copy(b0_h.at[pl.ds(nodew, _WPT)],
                            bb.at[pl.ds(0, _WPT)])

        nv = (_WPT + 15) // 16

        pltpu.sync_copy(stage.at[pl.ds(nodew, _WPT)], tmp.at[pl.ds(0, _WPT)])

        def cp(i, c):
            mrg[pl.ds(i * 16, 16)] = tmp[pl.ds(i * 16, 16)]
            return c

        lax.fori_loop(0, nv, cp, 0)

        def tmerge(t, c):
            pltpu.sync_copy(stage.at[pl.ds(t * (_N * _OC) + nodew, _WPT)],
                            tmp.at[pl.ds(0, _WPT)])

            def mx(i, c2):
                mrg[pl.ds(i * 16, 16)] = jnp.maximum(
                    mrg[pl.ds(i * 16, 16)], tmp[pl.ds(i * 16, 16)])
                return c2

            return lax.fori_loop(0, nv, mx, c)

        lax.fori_loop(1, _NT, tmerge, 0)

        def hfun(i, c):
            m = mrg[pl.ds(i * 16, 16)]
            a = jnp.where(m == _SENT, 0.0, m)
            hb[pl.ds(i * 16, 16)] = jnp.clip(
                a + bb[pl.ds(i * 16, 16)], 0.0, 6.0)
            return c

        lax.fori_loop(0, nv, hfun, 0)

        lm = jnp.minimum(l, _IT - 1)

        @pl.when(l < _IT)
        def _():
            def bfun(p, c):
                hv = hb[pl.ds(p * 16, 16)]
                b = bdv[lm, pl.ds(0, 16)]
                for i in range(_OC):
                    b = b + _vg(hv, idx01 * 8 + i) * rdv[lm, i, pl.ds(0, 16)]
                bb[pl.ds(p * 16, 16)] = b
                return c

            lax.fori_loop(0, nv, bfun, 0)

            for g in range(5):
                def zfun(r, c):
                    hv = hb[pl.ds((g * 125 + r) * 8, 16)]
                    a01 = jnp.zeros((16,), jnp.float32)
                    a23 = jnp.zeros((16,), jnp.float32)
                    ab = jnp.zeros((16,), jnp.float32)
                    for i in range(_OC):
                        s = _vg(hv, lane * 0 + i)
                        a01 = a01 + s * wzv[i, pl.ds(0, 16)]
                        a23 = a23 + s * wzv[i, pl.ds(16, 16)]
                        ab = ab + s * wzv[i, pl.ds(32, 16)]
                    zbuf[r, pl.ds(0, 16)] = a01
                    zbuf[r, pl.ds(16, 16)] = a23
                    zbuf[r, pl.ds(32, 16)] = ab
                    return c

                lax.fori_loop(0, 125, zfun, 0)
                pltpu.sync_copy(zbuf, zs_h.at[pl.ds(nbase + g * 125, 125)])

            plsc.subcore_barrier()

        @pl.when(l == _IT)
        def _():
            pltpu.sync_copy(hb.at[pl.ds(0, _WPT)],
                            hout_h.at[pl.ds(nodew, _WPT)])

        return carry

    lax.fori_loop(0, _IT + 1, layer_body, 0)


def _sc_gnn(src, dst, ea4, z0, b0, wz, rd, bd, sent):
    mesh = plsc.VectorSubcoreMesh(
        core_axis_name="c", subcore_axis_name="s", num_cores=1)
    f = functools.partial(
        pl.kernel,
        out_type=jax.ShapeDtypeStruct((_N * _OC,), jnp.float32),
        mesh=mesh,
        compiler_params=pltpu.CompilerParams(use_tc_tiling_on_sc=False),
        scratch_types=[
            pltpu.HBM((_N, 48), jnp.float32),
            pltpu.HBM((_NT * _N * _OC,), jnp.float32),
            pltpu.VMEM((_EPT * _OC + 16,), jnp.float32),
            pltpu.VMEM((_C,), jnp.int32),
            pltpu.VMEM((_C,), jnp.int32),
            pltpu.VMEM((_C,), jnp.int32),
            pltpu.VMEM((_C,), jnp.int32),
            pltpu.VMEM((_C,), jnp.int32),
            pltpu.VMEM((_C,), jnp.int32),
            pltpu.VMEM((_C,), jnp.int32),
            pltpu.VMEM((_C,), jnp.int32),
            pltpu.VMEM((_C * 4 + 16,), jnp.float32),
            pltpu.VMEM((_C * 4 + 16,), jnp.float32),
            pltpu.VMEM((_C * 4 + 16,), jnp.float32),
            pltpu.VMEM((_C * 4 + 16,), jnp.float32),
            pltpu.VMEM((_C, 48), jnp.float32),
            pltpu.VMEM((_C, 48), jnp.float32),
            pltpu.VMEM((_WPT + 16,), jnp.float32),
            pltpu.VMEM((_WPT + 16,), jnp.float32),
            pltpu.VMEM((_WPT + 16,), jnp.float32),
            pltpu.VMEM((_WPT + 16,), jnp.float32),
            pltpu.VMEM((125, 48), jnp.float32),
            pltpu.VMEM((_OC, 48), jnp.float32),
            pltpu.VMEM((_IT, _OC, 16), jnp.float32),
            pltpu.VMEM((_IT, 16), jnp.float32),
            pltpu.SemaphoreType.DMA,
            pltpu.SemaphoreType.DMA,
            pltpu.SemaphoreType.DMA,
            pltpu.SemaphoreType.DMA,
            pltpu.SemaphoreType.DMA,
            pltpu.SemaphoreType.DMA,
            pltpu.SemaphoreType.DMA,
        ],
    )(_gnn_body)
    return f(src, dst, ea4, z0, b0, wz, rd, bd, sent)


def kernel(x, edge_index, edge_attr, W_fe, b_fe, W_e, b_e, root0, bias0,
           roots, biases, W_o1, b_o1, W_o2, b_o2):
    f32 = jnp.float32
    src = edge_index[0].astype(jnp.int32)
    dst = edge_index[1].astype(jnp.int32)
    ea4 = edge_attr.reshape(-1)
    sent = jnp.full((_N * _OC + 16,), _SENT, f32)

    wfe_r = W_fe.reshape(_DE, _DF, _OC).transpose(1, 0, 2).reshape(_DF, 32)
    a56 = jnp.concatenate(
        [wfe_r, b_fe.reshape(_DF, _OC), jnp.zeros((_DF, 8), f32), root0],
        axis=1)
    bias56 = jnp.concatenate(
        [jnp.zeros((48,), f32), bias0]).reshape(1, 56)

    we_r = W_e.reshape(_DE, _OC, _OC).transpose(1, 0, 2).reshape(_OC, 32)
    wz = jnp.concatenate(
        [we_r, b_e.reshape(_OC, _OC), jnp.zeros((_OC, 8), f32)], axis=1)
    rd = jnp.concatenate([roots, roots], axis=2)
    bd = jnp.concatenate([biases, biases], axis=1)

    zb0 = _tc_prep(x, a56, bias56)
    z0 = zb0[:, :48]
    b0 = zb0[:, 48:].reshape(-1)

    h4 = _sc_gnn(src, dst, ea4, z0, b0, wz, rd, bd, sent)

    w1r = W_o1.reshape(_N, 128)
    r8 = jnp.zeros((_OC, 128), f32)
    r8 = r8.at[jnp.arange(8).repeat(16), jnp.arange(128)].set(1.0)
    smat = jnp.zeros((128, 16), f32)
    smat = smat.at[jnp.arange(128), jnp.arange(128) % 16].set(1.0)
    out = _tc_readout(h4.reshape(_N, _OC), w1r, r8,
                      smat, b_o1.reshape(1, 16), W_o2, b_o2.reshape(1, 1))
    return out.reshape(1)


# speedup vs baseline: 1.8692x; 1.0415x over previous
import functools

import jax
import jax.numpy as jnp
from jax import lax
from jax.experimental import pallas as pl
from jax.experimental.pallas import tpu as pltpu
from jax.experimental.pallas import tpu_sc as plsc

_N = 10000
_E = 160000
_DF = 128
_DE = 4
_OC = 8
_IT = 4

_NT = 16
_EPT = _E // _NT
_C = 80
_NCH = _EPT // _C
_NPT = _N // _NT
_WPT = _NPT * _OC
_SENT = -1.0e30

_GDN = lax.GatherDimensionNumbers(
    offset_dims=(), collapsed_slice_dims=(0,), start_index_map=(0,))


def _vg(v, idx):
    return lax.gather(v, idx[:, None], _GDN, (1,),
                      mode=lax.GatherScatterMode.PROMISE_IN_BOUNDS)


def _prep_body(x_ref, a_ref, b_ref, o_ref):
    o_ref[...] = (
        jnp.dot(x_ref[...], a_ref[...], preferred_element_type=jnp.float32,
                precision=lax.Precision.HIGHEST)
        + b_ref[...])


def _tc_prep(x, a56, bias56):
    return pl.pallas_call(
        _prep_body,
        grid=(10,),
        in_specs=[
            pl.BlockSpec((_N // 10, _DF), lambda i: (i, 0)),
            pl.BlockSpec((_DF, 56), lambda i: (0, 0)),
            pl.BlockSpec((1, 56), lambda i: (0, 0)),
        ],
        out_specs=pl.BlockSpec((_N // 10, 56), lambda i: (i, 0)),
        out_shape=jax.ShapeDtypeStruct((_N, 56), jnp.float32),
    )(x, a56, bias56)


def _readout_body(h_ref, w1_ref, r8_ref, s_ref, b1_ref, w2_ref, b2_ref,
                  o_ref, acc_ref):
    i = pl.program_id(0)

    @pl.when(i == 0)
    def _():
        acc_ref[...] = jnp.zeros_like(acc_ref)

    hrep = jnp.dot(h_ref[...], r8_ref[...],
                   preferred_element_type=jnp.float32,
                   precision=lax.Precision.HIGHEST)
    acc_ref[...] += jnp.sum(hrep * w1_ref[...], axis=0, keepdims=True)

    @pl.when(i == pl.num_programs(0) - 1)
    def _():
        s = jnp.dot(acc_ref[...], s_ref[...],
                    preferred_element_type=jnp.float32,
                    precision=lax.Precision.HIGHEST) + b1_ref[...]
        s = jnp.where(s > 0, s, jnp.exp(s) - 1.0)
        o = jnp.dot(s, w2_ref[...],
                    preferred_element_type=jnp.float32,
                    precision=lax.Precision.HIGHEST) + b2_ref[...]
        o_ref[...] = jnp.where(o > 0, o, jnp.exp(o) - 1.0)


def _tc_readout(h2d, w1r, r8, smat, b1, w2, b2):
    return pl.pallas_call(
        _readout_body,
        grid=(10,),
        in_specs=[
            pl.BlockSpec((_N // 10, _OC), lambda i: (i, 0)),
            pl.BlockSpec((_N // 10, 128), lambda i: (i, 0)),
            pl.BlockSpec((_OC, 128), lambda i: (0, 0)),
            pl.BlockSpec((128, 16), lambda i: (0, 0)),
            pl.BlockSpec((1, 16), lambda i: (0, 0)),
            pl.BlockSpec((16, 1), lambda i: (0, 0)),
            pl.BlockSpec((1, 1), lambda i: (0, 0)),
        ],
        out_specs=pl.BlockSpec((1, 1), lambda i: (0, 0)),
        out_shape=jax.ShapeDtypeStruct((1, 1), jnp.float32),
        scratch_shapes=[pltpu.VMEM((1, 128), jnp.float32)],
    )(h2d, w1r, r8, smat, b1, w2, b2)


def _gnn_body(src_h, dst_h, ea_h, z0_h, b0_h, wz_h, rd_h, bd_h, sent_h,
              hout_h,
              zs_h, stage, acc,
              srcv0, srcv1, srcv2, srcv3,
              dstv0, dstv1, dstv2, dstv3,
              eav0, eav1, eav2, eav3, zrv0, zrv1,
              mrg, tmp, tmp2, hb, bb, zbuf, wzv, rdv, bdv,
              semg0, semg1, semi, si0, si1, si2, si3):
    wid = lax.axis_index("s")
    ebase = wid * _EPT
    nodew = wid * _WPT
    nbase = wid * _NPT

    pltpu.sync_copy(wz_h, wzv)
    pltpu.sync_copy(rd_h, rdv)
    pltpu.sync_copy(bd_h, bdv)

    lane = lax.iota(jnp.int32, 16)
    lo8 = lane < 8
    idx01 = lane >> 3
    idx23 = idx01 + 2
    swap = lane ^ 8
    sentv = jnp.full((16,), _SENT, jnp.float32)

    pltpu.async_copy(sent_h, acc, semi)

    def load_idx(k, srcv, dstv, eav, sem):
        eb = ebase + k * _C
        pltpu.async_copy(src_h.at[pl.ds(eb, _C)], srcv, sem)
        pltpu.async_copy(dst_h.at[pl.ds(eb, _C)], dstv, sem)
        pltpu.async_copy(ea_h.at[pl.ds(eb * 4, _C * 4)],
                         eav.at[pl.ds(0, _C * 4)], sem)

    def wait_idx(srcv, dstv, eav, sem):
        pltpu.make_async_copy(src_h.at[pl.ds(0, _C)], srcv, sem).wait()
        pltpu.make_async_copy(dst_h.at[pl.ds(0, _C)], dstv, sem).wait()
        pltpu.make_async_copy(ea_h.at[pl.ds(0, _C * 4)],
                              eav.at[pl.ds(0, _C * 4)], sem).wait()

    for g in range(5):
        pltpu.sync_copy(z0_h.at[pl.ds(nbase + g * 125, 125)], zbuf)
        pltpu.sync_copy(zbuf, zs_h.at[pl.ds(nbase + g * 125, 125)])
    plsc.subcore_barrier()

    def layer_body(l, carry):
        zt = zs_h

        def compute_chunk(dstv, eav, zrv):
            def group_body(g, c2):
                d8vec = dstv[pl.ds(g * 16, 16)] * 8
                for j in range(16):
                    vea = eav[pl.ds(g * 64 + j * 4, 16)]
                    se01 = _vg(vea, idx01)
                    se23 = _vg(vea, idx23)
                    e = g * 16 + j
                    v01 = zrv[e, pl.ds(0, 16)]
                    v23 = zrv[e, pl.ds(16, 16)]
                    vb = zrv[e, pl.ds(32, 16)]
                    m = vb + se01 * v01 + se23 * v23
                    m = m + _vg(m, swap)
                    m = jnp.where(lo8, m, sentv)
                    d8 = d8vec[j]
                    acc[pl.ds(d8, 16)] = jnp.maximum(acc[pl.ds(d8, 16)], m)
                return c2

            lax.fori_loop(0, _C // 16, group_body, 0)

        load_idx(0, srcv0, dstv0, eav0, si0)
        wait_idx(srcv0, dstv0, eav0, si0)
        load_idx(1, srcv1, dstv1, eav1, si1)
        wait_idx(srcv1, dstv1, eav1, si1)
        pltpu.async_copy(zt.at[srcv0], zrv0, semg0)
        pltpu.async_copy(zt.at[srcv1], zrv1, semg1)
        load_idx(2, srcv2, dstv2, eav2, si2)
        load_idx(3, srcv3, dstv3, eav3, si3)
        pltpu.make_async_copy(sent_h, acc, semi).wait()

        idx_sets = ((srcv0, dstv0, eav0, si0), (srcv1, dstv1, eav1, si1),
                    (srcv2, dstv2, eav2, si2), (srcv3, dstv3, eav3, si3))

        def quad_body(q, c):
            k0 = 4 * q
            for s in range(4):
                k = k0 + s
                srcv, dstv, eav, si = idx_sets[s]
                nsrcv, ndstv, neav, nsi = idx_sets[(s + 2) % 4]
                zrv, semg = (zrv0, semg0) if s % 2 == 0 else (zrv1, semg1)

                if s == 0:
                    pltpu.make_async_copy(zt.at[srcv], zrv, semg).wait()
                    compute_chunk(dstv, eav, zrv)
                else:
                    @pl.when(k < _NCH)
                    def _():
                        pltpu.make_async_copy(zt.at[srcv], zrv, semg).wait()
                        compute_chunk(dstv, eav, zrv)

                @pl.when(k + 2 < _NCH)
                def _():
                    wait_idx(nsrcv, ndstv, neav, nsi)
                    pltpu.async_copy(zt.at[nsrcv], zrv, semg)

                @pl.when(k + 4 < _NCH)
                def _():
                    load_idx(k + 4, srcv, dstv, eav, si)

            return c

        lax.fori_loop(0, (_NCH + 3) // 4, quad_body, 0)

        pltpu.sync_copy(acc.at[pl.ds(0, _N * _OC)],
                        stage.at[pl.ds(wid * (_N * _OC), _N * _OC)])
        plsc.subcore_barrier()

        @pl.when(l < _IT)
        def _():
            pltpu.async_copy(sent_h, acc, semi)

        @pl.when(l == 0)
        def _():
            pltpu.sync_copy(b0_h.at[pl.ds(nodew, _WPT)],
                            bb.at[pl.ds(0, _WPT)])

        nv = (_WPT + 15) // 16

        pltpu.async_copy(stage.at[pl.ds(nodew, _WPT)],
                         tmp.at[pl.ds(0, _WPT)], si0)
        pltpu.async_copy(stage.at[pl.ds(_N * _OC + nodew, _WPT)],
                         tmp2.at[pl.ds(0, _WPT)], si1)
        pltpu.make_async_copy(stage.at[pl.ds(0, _WPT)],
                              tmp.at[pl.ds(0, _WPT)], si0).wait()

        def cp(i, c):
            mrg[pl.ds(i * 16, 16)] = tmp[pl.ds(i * 16, 16)]
            return c

        lax.fori_loop(0, nv, cp, 0)
        pltpu.async_copy(stage.at[pl.ds(2 * (_N * _OC) + nodew, _WPT)],
                         tmp.at[pl.ds(0, _WPT)], si0)

        def tmerge(p, c):
            for s in range(2):
                t = 2 * p + 1 + s
                tb, sb = (tmp2, si1) if s == 0 else (tmp, si0)

                @pl.when(t < _NT)
                def _():
                    pltpu.make_async_copy(stage.at[pl.ds(0, _WPT)],
                                          tb.at[pl.ds(0, _WPT)], sb).wait()

                    def mx(i, c2):
                        mrg[pl.ds(i * 16, 16)] = jnp.maximum(
                            mrg[pl.ds(i * 16, 16)], tb[pl.ds(i * 16, 16)])
                        return c2

                    lax.fori_loop(0, nv, mx, 0)

                    @pl.when(t + 2 < _NT)
                    def _():
                        pltpu.async_copy(
                            stage.at[pl.ds((t + 2) * (_N * _OC) + nodew,
                                           _WPT)],
                            tb.at[pl.ds(0, _WPT)], sb)

            return c

        lax.fori_loop(0, _NT // 2, tmerge, 0)

        def hfun(i, c):
            m = mrg[pl.ds(i * 16, 16)]
            a = jnp.where(m == _SENT, 0.0, m)
            hb[pl.ds(i * 16, 16)] = jnp.clip(
                a + bb[pl.ds(i * 16, 16)], 0.0, 6.0)
            return c

        lax.fori_loop(0, nv, hfun, 0)

        lm = jnp.minimum(l, _IT - 1)

        @pl.when(l < _IT)
        def _():
            def bfun(p, c):
                hv = hb[pl.ds(p * 16, 16)]
                b = bdv[lm, pl.ds(0, 16)]
                for i in range(_OC):
                    b = b + _vg(hv, idx01 * 8 + i) * rdv[lm, i, pl.ds(0, 16)]
                bb[pl.ds(p * 16, 16)] = b
                return c

            lax.fori_loop(0, nv, bfun, 0)

            for g in range(5):
                def zfun(r, c):
                    hv = hb[pl.ds((g * 125 + r) * 8, 16)]
                    a01 = jnp.zeros((16,), jnp.float32)
                    a23 = jnp.zeros((16,), jnp.float32)
                    ab = jnp.zeros((16,), jnp.float32)
                    for i in range(_OC):
                        s = _vg(hv, lane * 0 + i)
                        a01 = a01 + s * wzv[i, pl.ds(0, 16)]
                        a23 = a23 + s * wzv[i, pl.ds(16, 16)]
                        ab = ab + s * wzv[i, pl.ds(32, 16)]
                    zbuf[r, pl.ds(0, 16)] = a01
                    zbuf[r, pl.ds(16, 16)] = a23
                    zbuf[r, pl.ds(32, 16)] = ab
                    return c

                lax.fori_loop(0, 125, zfun, 0)
                pltpu.sync_copy(zbuf, zs_h.at[pl.ds(nbase + g * 125, 125)])

            plsc.subcore_barrier()

        @pl.when(l == _IT)
        def _():
            pltpu.sync_copy(hb.at[pl.ds(0, _WPT)],
                            hout_h.at[pl.ds(nodew, _WPT)])

        return carry

    lax.fori_loop(0, _IT + 1, layer_body, 0)


def _sc_gnn(src, dst, ea4, z0, b0, wz, rd, bd, sent):
    mesh = plsc.VectorSubcoreMesh(
        core_axis_name="c", subcore_axis_name="s", num_cores=1)
    f = functools.partial(
        pl.kernel,
        out_type=jax.ShapeDtypeStruct((_N * _OC,), jnp.float32),
        mesh=mesh,
        compiler_params=pltpu.CompilerParams(use_tc_tiling_on_sc=False),
        scratch_types=[
            pltpu.HBM((_N, 48), jnp.float32),
            pltpu.HBM((_NT * _N * _OC,), jnp.float32),
            pltpu.VMEM((_EPT * _OC + 16,), jnp.float32),
            pltpu.VMEM((_C,), jnp.int32),
            pltpu.VMEM((_C,), jnp.int32),
            pltpu.VMEM((_C,), jnp.int32),
            pltpu.VMEM((_C,), jnp.int32),
            pltpu.VMEM((_C,), jnp.int32),
            pltpu.VMEM((_C,), jnp.int32),
            pltpu.VMEM((_C,), jnp.int32),
            pltpu.VMEM((_C,), jnp.int32),
            pltpu.VMEM((_C * 4 + 16,), jnp.float32),
            pltpu.VMEM((_C * 4 + 16,), jnp.float32),
            pltpu.VMEM((_C * 4 + 16,), jnp.float32),
            pltpu.VMEM((_C * 4 + 16,), jnp.float32),
            pltpu.VMEM((_C, 48), jnp.float32),
            pltpu.VMEM((_C, 48), jnp.float32),
            pltpu.VMEM((_WPT + 16,), jnp.float32),
            pltpu.VMEM((_WPT + 16,), jnp.float32),
            pltpu.VMEM((_WPT + 16,), jnp.float32),
            pltpu.VMEM((_WPT + 16,), jnp.float32),
            pltpu.VMEM((_WPT + 16,), jnp.float32),
            pltpu.VMEM((125, 48), jnp.float32),
            pltpu.VMEM((_OC, 48), jnp.float32),
            pltpu.VMEM((_IT, _OC, 16), jnp.float32),
            pltpu.VMEM((_IT, 16), jnp.float32),
            pltpu.SemaphoreType.DMA,
            pltpu.SemaphoreType.DMA,
            pltpu.SemaphoreType.DMA,
            pltpu.SemaphoreType.DMA,
            pltpu.SemaphoreType.DMA,
            pltpu.SemaphoreType.DMA,
            pltpu.SemaphoreType.DMA,
        ],
    )(_gnn_body)
    return f(src, dst, ea4, z0, b0, wz, rd, bd, sent)


def kernel(x, edge_index, edge_attr, W_fe, b_fe, W_e, b_e, root0, bias0,
           roots, biases, W_o1, b_o1, W_o2, b_o2):
    f32 = jnp.float32
    src = edge_index[0].astype(jnp.int32)
    dst = edge_index[1].astype(jnp.int32)
    ea4 = edge_attr.reshape(-1)
    sent = jnp.full((_N * _OC + 16,), _SENT, f32)

    wfe_r = W_fe.reshape(_DE, _DF, _OC).transpose(1, 0, 2).reshape(_DF, 32)
    a56 = jnp.concatenate(
        [wfe_r, b_fe.reshape(_DF, _OC), jnp.zeros((_DF, 8), f32), root0],
        axis=1)
    bias56 = jnp.concatenate(
        [jnp.zeros((48,), f32), bias0]).reshape(1, 56)

    we_r = W_e.reshape(_DE, _OC, _OC).transpose(1, 0, 2).reshape(_OC, 32)
    wz = jnp.concatenate(
        [we_r, b_e.reshape(_OC, _OC), jnp.zeros((_OC, 8), f32)], axis=1)
    rd = jnp.concatenate([roots, roots], axis=2)
    bd = jnp.concatenate([biases, biases], axis=1)

    zb0 = _tc_prep(x, a56, bias56)
    z0 = zb0[:, :48]
    b0 = zb0[:, 48:].reshape(-1)

    h4 = _sc_gnn(src, dst, ea4, z0, b0, wz, rd, bd, sent)

    w1r = W_o1.reshape(_N, 128)
    r8 = jnp.zeros((_OC, 128), f32)
    r8 = r8.at[jnp.arange(8).repeat(16), jnp.arange(128)].set(1.0)
    smat = jnp.zeros((128, 16), f32)
    smat = smat.at[jnp.arange(128), jnp.arange(128) % 16].set(1.0)
    out = _tc_readout(h4.reshape(_N, _OC), w1r, r8,
                      smat, b_o1.reshape(1, 16), W_o2, b_o2.reshape(1, 1))
    return out.reshape(1)
